# Initial kernel scaffold; baseline (speedup 1.0000x reference)
import jax
import jax.numpy as jnp
from jax.experimental import pallas as pl


def kernel(adj, user_emb, item_emb, image_emb, text_emb, W_img, b_img, W_txt, b_txt, modal_weight, image_original_adj, text_original_adj):
    raise NotImplementedError("write your pallas kernel here")


# speedup vs baseline: 3.9159x; 3.9159x over previous
import functools

import jax
import jax.numpy as jnp
from jax import lax
from jax.experimental import pallas as pl
from jax.experimental.pallas import tpu as pltpu

N_USERS = 8192
N_ITEMS = 2048
EMBED = 64
TOPK = 10
LAMBDA = 0.9

_INTERPRET = False


def _feats_body(img_ref, txt_ref, wi_ref, bi_ref, wt_ref, bt_ref,
                xni_ref, xnt_ref):
    fi = jnp.dot(img_ref[...], wi_ref[...],
                 preferred_element_type=jnp.float32) + bi_ref[...]
    ni = jnp.sqrt(jnp.sum(fi * fi, axis=1, keepdims=True))
    xni_ref[...] = fi / ni
    ft = jnp.dot(txt_ref[...], wt_ref[...],
                 preferred_element_type=jnp.float32) + bt_ref[...]
    nt = jnp.sqrt(jnp.sum(ft * ft, axis=1, keepdims=True))
    xnt_ref[...] = ft / nt


def _top10_threshold(s):
    colid = lax.broadcasted_iota(jnp.int32, s.shape, 1)
    tot = jnp.zeros((s.shape[0], 1), jnp.float32)
    kth = None
    for _ in range(TOPK):
        m = jnp.max(s, axis=1, keepdims=True)
        idx = jnp.min(jnp.where(s == m, colid, s.shape[1]), axis=1,
                      keepdims=True)
        tot = tot + m
        s = jnp.where(colid == idx, -jnp.inf, s)
        kth = m
    return tot, kth


def _topk_body(xi_ref, xif_ref, xt_ref, xtf_ref, w_ref,
               kthi_ref, ktht_ref, dis_ref):
    dn = (((1,), (1,)), ((), ()))
    simi = lax.dot_general(xi_ref[...], xif_ref[...], dn,
                           preferred_element_type=jnp.float32)
    simt = lax.dot_general(xt_ref[...], xtf_ref[...], dn,
                           preferred_element_type=jnp.float32)
    ti, ki = _top10_threshold(simi)
    tt, kt = _top10_threshold(simt)
    w0 = w_ref[0, 0]
    w1 = w_ref[0, 1]
    rs = w0 * ti + w1 * tt
    di = lax.rsqrt(rs)
    di = jnp.where(jnp.isinf(di), 0.0, di)
    kthi_ref[...] = ki
    ktht_ref[...] = kt
    dis_ref[...] = di


def _item_body(xi_ref, xif_ref, xt_ref, xtf_ref, kthi_ref, ktht_ref,
               disr_ref, disc_ref, io_ref, to_ref, emb_ref, w_ref, hn_ref):
    dn = (((1,), (1,)), ((), ()))
    simi = lax.dot_general(xi_ref[...], xif_ref[...], dn,
                           preferred_element_type=jnp.float32)
    simt = lax.dot_general(xt_ref[...], xtf_ref[...], dn,
                           preferred_element_type=jnp.float32)
    mi = jnp.where(simi >= kthi_ref[...], simi, 0.0)
    mt = jnp.where(simt >= ktht_ref[...], simt, 0.0)
    w0 = w_ref[0, 0]
    w1 = w_ref[0, 1]
    learned = (w0 * mi + w1 * mt) * disr_ref[...] * disc_ref[...]
    a = LAMBDA * (w0 * io_ref[...] + w1 * to_ref[...]) + (1.0 - LAMBDA) * learned
    h = jnp.dot(a, emb_ref[...], preferred_element_type=jnp.float32)
    n = jnp.sqrt(jnp.sum(h * h, axis=1, keepdims=True))
    hn_ref[...] = h / jnp.maximum(n, 1e-12)


def _mm_body(a_ref, b_ref, o_ref):
    @pl.when(pl.program_id(1) == 0)
    def _():
        o_ref[...] = jnp.zeros_like(o_ref)

    o_ref[...] += jnp.dot(a_ref[...], b_ref[...],
                          preferred_element_type=jnp.float32)


def _mm_final_body(a_ref, b_ref, e0_ref, e1m_ref, h_ref, o_ref, *, nk, ni_blk):
    k = pl.program_id(1)
    i = pl.program_id(0)
    mask = jnp.where(i >= ni_blk, 1.0, 0.0)

    @pl.when(k == 0)
    def _():
        o_ref[...] = jnp.zeros_like(o_ref)

    o_ref[...] += jnp.dot(a_ref[...], b_ref[...],
                          preferred_element_type=jnp.float32)

    @pl.when(k == nk - 1)
    def _():
        r = (o_ref[...] + e0_ref[...] + e1m_ref[...]) * (1.0 / 3.0)
        o_ref[...] = r + mask * h_ref[...]


def kernel(adj, user_emb, item_emb, image_emb, text_emb, W_img, b_img,
           W_txt, b_txt, modal_weight, image_original_adj, text_original_adj):
    f32 = jnp.float32
    w = jax.nn.softmax(modal_weight, axis=0)
    warr = w.reshape(1, 2)

    xni, xnt = pl.pallas_call(
        _feats_body,
        out_shape=[jax.ShapeDtypeStruct((N_ITEMS, EMBED), f32),
                   jax.ShapeDtypeStruct((N_ITEMS, EMBED), f32)],
        interpret=_INTERPRET,
    )(image_emb, text_emb, W_img, b_img.reshape(1, EMBED),
      W_txt, b_txt.reshape(1, EMBED))

    bm2 = 256
    g2 = N_ITEMS // bm2
    row_spec = pl.BlockSpec((bm2, EMBED), lambda i: (i, 0))
    full_spec = pl.BlockSpec((N_ITEMS, EMBED), lambda i: (0, 0))
    w_spec = pl.BlockSpec((1, 2), lambda i: (0, 0))
    col1_spec = pl.BlockSpec((bm2, 1), lambda i: (i, 0))
    kthi, ktht, dis = pl.pallas_call(
        _topk_body,
        grid=(g2,),
        in_specs=[row_spec, full_spec, row_spec, full_spec, w_spec],
        out_specs=[col1_spec, col1_spec, col1_spec],
        out_shape=[jax.ShapeDtypeStruct((N_ITEMS, 1), f32),
                   jax.ShapeDtypeStruct((N_ITEMS, 1), f32),
                   jax.ShapeDtypeStruct((N_ITEMS, 1), f32)],
        interpret=_INTERPRET,
    )(xni, xni, xnt, xnt, warr)

    bm3 = 256
    g3 = N_ITEMS // bm3
    row3 = pl.BlockSpec((bm3, EMBED), lambda i: (i, 0))
    full3 = pl.BlockSpec((N_ITEMS, EMBED), lambda i: (0, 0))
    col13 = pl.BlockSpec((bm3, 1), lambda i: (i, 0))
    rowN3 = pl.BlockSpec((bm3, N_ITEMS), lambda i: (i, 0))
    disc_spec = pl.BlockSpec((1, N_ITEMS), lambda i: (0, 0))
    w3 = pl.BlockSpec((1, 2), lambda i: (0, 0))
    h_norm = pl.pallas_call(
        _item_body,
        grid=(g3,),
        in_specs=[row3, full3, row3, full3, col13, col13, col13,
                  disc_spec, rowN3, rowN3, full3, w3],
        out_specs=row3,
        out_shape=jax.ShapeDtypeStruct((N_ITEMS, EMBED), f32),
        interpret=_INTERPRET,
    )(xni, xni, xnt, xnt, kthi, ktht, dis, dis.reshape(1, N_ITEMS),
      image_original_adj, text_original_adj, item_emb, warr)

    n = N_USERS + N_ITEMS
    ego0 = jnp.concatenate([user_emb, item_emb], axis=0)
    bm, bk = 256, 2048
    nm, nk = n // bm, n // bk
    a_spec = pl.BlockSpec((bm, bk), lambda i, k: (i, k))
    b_spec = pl.BlockSpec((bk, EMBED), lambda i, k: (k, 0))
    o_spec = pl.BlockSpec((bm, EMBED), lambda i, k: (i, 0))
    e1 = pl.pallas_call(
        _mm_body,
        grid=(nm, nk),
        in_specs=[a_spec, b_spec],
        out_specs=o_spec,
        out_shape=jax.ShapeDtypeStruct((n, EMBED), f32),
        compiler_params=pltpu.CompilerParams(
            dimension_semantics=("parallel", "arbitrary")),
        interpret=_INTERPRET,
    )(adj, ego0)

    ni_blk = N_USERS // bm
    h_spec = pl.BlockSpec(
        (bm, EMBED), lambda i, k: (jnp.maximum(i - ni_blk, 0), 0))
    out = pl.pallas_call(
        functools.partial(_mm_final_body, nk=nk, ni_blk=ni_blk),
        grid=(nm, nk),
        in_specs=[a_spec, b_spec, o_spec, o_spec, h_spec],
        out_specs=o_spec,
        out_shape=jax.ShapeDtypeStruct((n, EMBED), f32),
        compiler_params=pltpu.CompilerParams(
            dimension_semantics=("parallel", "arbitrary")),
        interpret=_INTERPRET,
    )(adj, e1, ego0, e1, h_norm)

    return (out[:N_USERS], out[N_USERS:])
